# baseline (speedup 1.0000x reference)
import jax
import jax.numpy as jnp
from jax.experimental import pallas as pl
from jax.experimental.pallas import tpu as pltpu

_UNITS = 1024
_MEMSIZE = 32
_NUMHEADS = 16
_HEADSIZE = _UNITS // _NUMHEADS
_DEPTH = 8
_EPS = 1e-3
_BB = 16
_NI = 8
_NO = 4


def _softmax_m(logits):
    mx = jnp.max(logits, axis=1, keepdims=True)
    e = jnp.exp(logits - mx)
    return e / jnp.sum(e, axis=1, keepdims=True)


def _expand_heads(w, eye_ref):
    b, m, h = w.shape
    we = jnp.dot(w.reshape(b * m, h), eye_ref[...],
                 preferred_element_type=jnp.float32)
    return we.reshape(b, m, _UNITS)


def _dnc_kernel(inputs_ref, state_ref, eye_ref, rw_w_ref, read_b_ref,
                write_b_ref, kern_w_ref, kern_b_ref, proj_w_ref,
                proj_b_ref, ln_g_ref, ln_b_ref, ro_w_ref, ro_b_ref,
                out_ref, memout_ref, mem_all_ref, mean_all_ref,
                kern_buf, proj_buf, sem_k, sem_p):
    l = pl.program_id(1)
    i = pl.program_id(2)
    mem_ref = mem_all_ref.at[i]
    mean_ref = mean_all_ref.at[i]
    bb = _BB
    slot = jax.lax.rem(l, 2)

    @pl.when(i == 0)
    def _weights():
        @pl.when(l == 0)
        def _():
            pltpu.make_async_copy(kern_w_ref.at[0], kern_buf.at[0],
                                  sem_k.at[0]).start()
            pltpu.make_async_copy(proj_w_ref.at[0], proj_buf.at[0],
                                  sem_p.at[0]).start()
        pltpu.make_async_copy(kern_w_ref.at[l], kern_buf.at[slot],
                              sem_k.at[slot]).wait()
        pltpu.make_async_copy(proj_w_ref.at[l], proj_buf.at[slot],
                              sem_p.at[slot]).wait()

        @pl.when(l < _DEPTH - 1)
        def _():
            nslot = 1 - slot
            pltpu.make_async_copy(kern_w_ref.at[l + 1], kern_buf.at[nslot],
                                  sem_k.at[nslot]).start()
            pltpu.make_async_copy(proj_w_ref.at[l + 1], proj_buf.at[nslot],
                                  sem_p.at[nslot]).start()

    @pl.when(l == 0)
    def _init():
        comb = inputs_ref[...] + state_ref[...]
        m3 = jnp.concatenate(
            [comb[:, h] for h in range(_NUMHEADS)], axis=-1)
        mem_ref[...] = m3
        mean_ref[...] = jnp.mean(m3, axis=1)

    @pl.when(l == _DEPTH // 2)
    def _mid():
        for p in range(2):
            h4 = mem_ref[:, p * 16:(p + 1) * 16, :].reshape(
                bb, 16, _NUMHEADS, _HEADSIZE)
            mem_ref[:, p * 16:(p + 1) * 16, :] = jnp.concatenate(
                [h4[:, h] for h in range(_NUMHEADS)], axis=-1)

    mem = mem_ref[...]
    mem2d = mem.reshape(bb * _MEMSIZE, _UNITS)
    mean = mean_ref[...]

    rw = rw_w_ref[0]
    logits2 = jnp.dot(mem2d, rw, preferred_element_type=jnp.float32)
    logits2 = logits2.reshape(bb, _MEMSIZE, 2 * _NUMHEADS)
    corr2 = jnp.dot(mean, rw, preferred_element_type=jnp.float32)

    rlog = logits2[..., :_NUMHEADS] \
        + (corr2[:, :_NUMHEADS] + read_b_ref[0])[:, None, :]
    w = _expand_heads(_softmax_m(rlog), eye_ref)
    att = jnp.sum(w * mem, axis=1)

    v = jnp.maximum(
        jnp.dot(att, kern_buf[slot], preferred_element_type=jnp.float32)
        + kern_b_ref[0], 0.0)
    v = jnp.dot(v, proj_buf[slot], preferred_element_type=jnp.float32) \
        + proj_b_ref[0]
    mu = jnp.mean(v, axis=-1, keepdims=True)
    var = jnp.mean(jnp.square(v - mu), axis=-1, keepdims=True)
    v = (v - mu) * jax.lax.rsqrt(var + _EPS) * ln_g_ref[...] + ln_b_ref[...]

    vcorr = jnp.dot(v, rw[:, _NUMHEADS:],
                    preferred_element_type=jnp.float32)
    wlog = logits2[..., _NUMHEADS:] \
        + (corr2[:, _NUMHEADS:] + vcorr + write_b_ref[0])[:, None, :]
    ww = _expand_heads(_softmax_m(wlog), eye_ref)
    newmem = mem + ww * (v[:, None, :] - mem)
    mem_ref[...] = newmem
    mean_ref[...] = jnp.mean(newmem, axis=1)

    @pl.when(l == _DEPTH - 1)
    def _final():
        nm2d = newmem.reshape(bb * _MEMSIZE, _UNITS)
        nmean = jnp.mean(newmem, axis=1)
        logits = jnp.dot(nm2d, ro_w_ref[...],
                         preferred_element_type=jnp.float32)
        corr = jnp.dot(nmean, ro_w_ref[...],
                       preferred_element_type=jnp.float32) + ro_b_ref[...]
        logits = logits.reshape(bb, _MEMSIZE, _NUMHEADS) + corr[:, None, :]
        w = _expand_heads(_softmax_m(logits), eye_ref)
        out_ref[...] = jnp.sum(w * newmem, axis=1)
        for s in range(_MEMSIZE):
            memout_ref[:, s, :] = newmem[:, (s % 2) * 16 + s // 2, :]


def kernel(inputs, state, read_W, read_b, write_W, write_b, kern_W, kern_b,
           proj_W, proj_b, ln_gamma, ln_beta, readout_W, readout_b):
    B = inputs.shape[0]

    in4 = inputs.reshape(B, _NUMHEADS, _MEMSIZE, _HEADSIZE)
    st4 = state.reshape(B, _NUMHEADS, _MEMSIZE, _HEADSIZE)
    rw_W = jnp.concatenate([read_W, write_W], axis=-1)
    read_b3 = read_b.reshape(_DEPTH, 1, _NUMHEADS)
    write_b3 = write_b.reshape(_DEPTH, 1, _NUMHEADS)
    kern_b3 = kern_b.reshape(_DEPTH, 1, _UNITS)
    proj_b3 = proj_b.reshape(_DEPTH, 1, _UNITS)
    ln_g2 = ln_gamma.reshape(1, _UNITS)
    ln_b2 = ln_beta.reshape(1, _UNITS)
    ro_b2 = readout_b.reshape(1, _NUMHEADS)
    eye = jnp.equal(
        jax.lax.broadcasted_iota(jnp.int32, (_NUMHEADS, _UNITS), 1)
        // _HEADSIZE,
        jax.lax.broadcasted_iota(jnp.int32, (_NUMHEADS, _UNITS), 0)
    ).astype(jnp.float32)

    bin_ = lambda o, l, i: (o * _NI + jnp.where(l == 0, i, _NI - 1), 0, 0, 0)
    bout2 = lambda o, l, i: (o * _NI + jnp.where(l == _DEPTH - 1, i, 0), 0)
    bout3 = lambda o, l, i: (o * _NI + jnp.where(l == _DEPTH - 1, i, 0), 0, 0)
    li = lambda o, l, i: (l, 0, 0)
    fixed2 = lambda o, l, i: (0, 0)

    out, memout = pl.pallas_call(
        _dnc_kernel,
        grid=(_NO, _DEPTH, _NI),
        in_specs=[
            pl.BlockSpec((_BB, _NUMHEADS, _MEMSIZE, _HEADSIZE), bin_),
            pl.BlockSpec((_BB, _NUMHEADS, _MEMSIZE, _HEADSIZE), bin_),
            pl.BlockSpec((_NUMHEADS, _UNITS), fixed2),
            pl.BlockSpec((1, _UNITS, 2 * _NUMHEADS), li),
            pl.BlockSpec((1, 1, _NUMHEADS), li),
            pl.BlockSpec((1, 1, _NUMHEADS), li),
            pl.BlockSpec(memory_space=pl.ANY),
            pl.BlockSpec((1, 1, _UNITS), li),
            pl.BlockSpec(memory_space=pl.ANY),
            pl.BlockSpec((1, 1, _UNITS), li),
            pl.BlockSpec((1, _UNITS), fixed2),
            pl.BlockSpec((1, _UNITS), fixed2),
            pl.BlockSpec((_UNITS, _NUMHEADS), fixed2),
            pl.BlockSpec((1, _NUMHEADS), fixed2),
        ],
        out_specs=[
            pl.BlockSpec((_BB, _UNITS), bout2),
            pl.BlockSpec((_BB, _MEMSIZE, _UNITS), bout3),
        ],
        out_shape=[
            jax.ShapeDtypeStruct((B, _UNITS), jnp.float32),
            jax.ShapeDtypeStruct((B, _MEMSIZE, _UNITS), jnp.float32),
        ],
        scratch_shapes=[
            pltpu.VMEM((_NI, _BB, _MEMSIZE, _UNITS), jnp.float32),
            pltpu.VMEM((_NI, _BB, _UNITS), jnp.float32),
            pltpu.VMEM((2, _UNITS, _UNITS), jnp.float32),
            pltpu.VMEM((2, _UNITS, _UNITS), jnp.float32),
            pltpu.SemaphoreType.DMA((2,)),
            pltpu.SemaphoreType.DMA((2,)),
        ],
        compiler_params=pltpu.CompilerParams(
            dimension_semantics=("parallel", "arbitrary", "arbitrary"),
            vmem_limit_bytes=61440 * 1024,
        ),
        name="dnccell",
    )(in4, st4, eye, rw_W, read_b3, write_b3, kern_W, kern_b3, proj_W,
      proj_b3, ln_g2, ln_b2, readout_W, ro_b2)
    return out, memout.reshape(B, _MEMSIZE * _UNITS)


# speedup vs baseline: 1.2550x; 1.2550x over previous
import jax
import jax.numpy as jnp
from jax.experimental import pallas as pl
from jax.experimental.pallas import tpu as pltpu

_UNITS = 1024
_MEMSIZE = 32
_NUMHEADS = 16
_HEADSIZE = _UNITS // _NUMHEADS
_DEPTH = 8
_EPS = 1e-3
_BB = 16
_NI = 8
_NO = 4


def _softmax_m(logits):
    mx = jnp.max(logits, axis=1, keepdims=True)
    e = jnp.exp(logits - mx)
    return e / jnp.sum(e, axis=1, keepdims=True)


def _expand_heads(w, eye_ref):
    b, m, h = w.shape
    we = jnp.dot(w.reshape(b * m, h), eye_ref[...],
                 preferred_element_type=jnp.float32)
    return we.reshape(b, m, _UNITS)


def _dnc_kernel(inputs_ref, state_ref, eye_ref, rw_w_ref, read_b_ref,
                write_b_ref, kern_w_ref, kern_b_ref, proj_w_ref,
                proj_b_ref, ln_g_ref, ln_b_ref, ro_w_ref, ro_b_ref,
                out_ref, memout_ref, mem_all_ref, mean_all_ref,
                kern_buf, proj_buf, sem_k, sem_p):
    l = pl.program_id(1)
    i = pl.program_id(2)
    mem_ref = mem_all_ref.at[i]
    mean_ref = mean_all_ref.at[i]
    bb = _BB
    slot = jax.lax.rem(l, 2)

    @pl.when(i == 0)
    def _weights():
        @pl.when(l == 0)
        def _():
            pltpu.make_async_copy(kern_w_ref.at[0], kern_buf.at[0],
                                  sem_k.at[0]).start()
            pltpu.make_async_copy(proj_w_ref.at[0], proj_buf.at[0],
                                  sem_p.at[0]).start()
        pltpu.make_async_copy(kern_w_ref.at[l], kern_buf.at[slot],
                              sem_k.at[slot]).wait()
        pltpu.make_async_copy(proj_w_ref.at[l], proj_buf.at[slot],
                              sem_p.at[slot]).wait()

        @pl.when(l < _DEPTH - 1)
        def _():
            nslot = 1 - slot
            pltpu.make_async_copy(kern_w_ref.at[l + 1], kern_buf.at[nslot],
                                  sem_k.at[nslot]).start()
            pltpu.make_async_copy(proj_w_ref.at[l + 1], proj_buf.at[nslot],
                                  sem_p.at[nslot]).start()

    @pl.when(l == 0)
    def _init():
        comb = inputs_ref[...] + state_ref[...]
        c4 = comb.reshape(bb, _NUMHEADS, _MEMSIZE, _HEADSIZE)
        m3 = jnp.concatenate(
            [c4[:, h] for h in range(_NUMHEADS)], axis=-1)
        mem_ref[...] = m3
        mean_ref[...] = jnp.mean(m3, axis=1)

    @pl.when(l == _DEPTH // 2)
    def _mid():
        for p in range(2):
            h4 = mem_ref[:, p * 16:(p + 1) * 16, :].reshape(
                bb, 16, _NUMHEADS, _HEADSIZE)
            mem_ref[:, p * 16:(p + 1) * 16, :] = jnp.concatenate(
                [h4[:, h] for h in range(_NUMHEADS)], axis=-1)

    mem = mem_ref[...]
    mem2d = mem.reshape(bb * _MEMSIZE, _UNITS)
    mean = mean_ref[...]

    rw = rw_w_ref[0]
    logits2 = jnp.dot(mem2d, rw, preferred_element_type=jnp.float32)
    logits2 = logits2.reshape(bb, _MEMSIZE, 2 * _NUMHEADS)
    corr2 = jnp.dot(mean, rw, preferred_element_type=jnp.float32)

    rlog = logits2[..., :_NUMHEADS] \
        + (corr2[:, :_NUMHEADS] + read_b_ref[0])[:, None, :]
    w = _expand_heads(_softmax_m(rlog), eye_ref)
    att = jnp.sum(w * mem, axis=1)

    v = jnp.maximum(
        jnp.dot(att, kern_buf[slot], preferred_element_type=jnp.float32)
        + kern_b_ref[0], 0.0)
    v = jnp.dot(v, proj_buf[slot], preferred_element_type=jnp.float32) \
        + proj_b_ref[0]
    mu = jnp.mean(v, axis=-1, keepdims=True)
    var = jnp.mean(jnp.square(v - mu), axis=-1, keepdims=True)
    v = (v - mu) * jax.lax.rsqrt(var + _EPS) * ln_g_ref[...] + ln_b_ref[...]

    vcorr = jnp.dot(v, rw[:, _NUMHEADS:],
                    preferred_element_type=jnp.float32)
    wlog = logits2[..., _NUMHEADS:] \
        + (corr2[:, _NUMHEADS:] + vcorr + write_b_ref[0])[:, None, :]
    ww = _expand_heads(_softmax_m(wlog), eye_ref)
    newmem = mem + ww * (v[:, None, :] - mem)
    mem_ref[...] = newmem
    mean_ref[...] = jnp.mean(newmem, axis=1)

    @pl.when(l == _DEPTH - 1)
    def _final():
        nm2d = newmem.reshape(bb * _MEMSIZE, _UNITS)
        nmean = jnp.mean(newmem, axis=1)
        logits = jnp.dot(nm2d, ro_w_ref[...],
                         preferred_element_type=jnp.float32)
        corr = jnp.dot(nmean, ro_w_ref[...],
                       preferred_element_type=jnp.float32) + ro_b_ref[...]
        logits = logits.reshape(bb, _MEMSIZE, _NUMHEADS) + corr[:, None, :]
        w = _expand_heads(_softmax_m(logits), eye_ref)
        out_ref[...] = jnp.sum(w * newmem, axis=1)
        for s in range(_MEMSIZE):
            memout_ref[:, s, :] = newmem[:, (s % 2) * 16 + s // 2, :]


def kernel(inputs, state, read_W, read_b, write_W, write_b, kern_W, kern_b,
           proj_W, proj_b, ln_gamma, ln_beta, readout_W, readout_b):
    B = inputs.shape[0]

    rw_W = jnp.concatenate([read_W, write_W], axis=-1)
    read_b3 = read_b.reshape(_DEPTH, 1, _NUMHEADS)
    write_b3 = write_b.reshape(_DEPTH, 1, _NUMHEADS)
    kern_b3 = kern_b.reshape(_DEPTH, 1, _UNITS)
    proj_b3 = proj_b.reshape(_DEPTH, 1, _UNITS)
    ln_g2 = ln_gamma.reshape(1, _UNITS)
    ln_b2 = ln_beta.reshape(1, _UNITS)
    ro_b2 = readout_b.reshape(1, _NUMHEADS)
    eye = jnp.equal(
        jax.lax.broadcasted_iota(jnp.int32, (_NUMHEADS, _UNITS), 1)
        // _HEADSIZE,
        jax.lax.broadcasted_iota(jnp.int32, (_NUMHEADS, _UNITS), 0)
    ).astype(jnp.float32)

    bin_ = lambda o, l, i: (o * _NI + jnp.where(l == 0, i, _NI - 1), 0)
    bout2 = lambda o, l, i: (o * _NI + jnp.where(l == _DEPTH - 1, i, 0), 0)
    bout3 = lambda o, l, i: (o * _NI + jnp.where(l == _DEPTH - 1, i, 0), 0, 0)
    li = lambda o, l, i: (l, 0, 0)
    fixed2 = lambda o, l, i: (0, 0)

    out, memout = pl.pallas_call(
        _dnc_kernel,
        grid=(_NO, _DEPTH, _NI),
        in_specs=[
            pl.BlockSpec((_BB, _MEMSIZE * _UNITS), bin_),
            pl.BlockSpec((_BB, _MEMSIZE * _UNITS), bin_),
            pl.BlockSpec((_NUMHEADS, _UNITS), fixed2),
            pl.BlockSpec((1, _UNITS, 2 * _NUMHEADS), li),
            pl.BlockSpec((1, 1, _NUMHEADS), li),
            pl.BlockSpec((1, 1, _NUMHEADS), li),
            pl.BlockSpec(memory_space=pl.ANY),
            pl.BlockSpec((1, 1, _UNITS), li),
            pl.BlockSpec(memory_space=pl.ANY),
            pl.BlockSpec((1, 1, _UNITS), li),
            pl.BlockSpec((1, _UNITS), fixed2),
            pl.BlockSpec((1, _UNITS), fixed2),
            pl.BlockSpec((_UNITS, _NUMHEADS), fixed2),
            pl.BlockSpec((1, _NUMHEADS), fixed2),
        ],
        out_specs=[
            pl.BlockSpec((_BB, _UNITS), bout2),
            pl.BlockSpec((_BB, _MEMSIZE, _UNITS), bout3),
        ],
        out_shape=[
            jax.ShapeDtypeStruct((B, _UNITS), jnp.float32),
            jax.ShapeDtypeStruct((B, _MEMSIZE, _UNITS), jnp.float32),
        ],
        scratch_shapes=[
            pltpu.VMEM((_NI, _BB, _MEMSIZE, _UNITS), jnp.float32),
            pltpu.VMEM((_NI, _BB, _UNITS), jnp.float32),
            pltpu.VMEM((2, _UNITS, _UNITS), jnp.float32),
            pltpu.VMEM((2, _UNITS, _UNITS), jnp.float32),
            pltpu.SemaphoreType.DMA((2,)),
            pltpu.SemaphoreType.DMA((2,)),
        ],
        compiler_params=pltpu.CompilerParams(
            dimension_semantics=("parallel", "arbitrary", "arbitrary"),
            vmem_limit_bytes=61440 * 1024,
        ),
        name="dnccell",
    )(inputs, state, eye, rw_W, read_b3, write_b3, kern_W, kern_b3, proj_W,
      proj_b3, ln_g2, ln_b2, readout_W, ro_b2)
    return out, memout.reshape(B, _MEMSIZE * _UNITS)


# speedup vs baseline: 1.3243x; 1.0552x over previous
import jax
import jax.numpy as jnp
from jax.experimental import pallas as pl
from jax.experimental.pallas import tpu as pltpu

_UNITS = 1024
_MEMSIZE = 32
_NUMHEADS = 16
_HEADSIZE = _UNITS // _NUMHEADS
_DEPTH = 8
_EPS = 1e-3
_BB = 16
_NI = 8
_NO = 4


def _softmax_m(logits):
    mx = jnp.max(logits, axis=1, keepdims=True)
    e = jnp.exp(logits - mx)
    return e / jnp.sum(e, axis=1, keepdims=True)


def _expand_heads(w, eye_ref):
    b, m, h = w.shape
    we = jnp.dot(w.reshape(b * m, h), eye_ref[...],
                 preferred_element_type=jnp.float32)
    return we.reshape(b, m, _UNITS)


def _dnc_kernel(inputs_ref, state_ref, eye_ref, rw_w_ref, read_b_ref,
                write_b_ref, kern_w_ref, kern_b_ref, proj_w_ref,
                proj_b_ref, ln_g_ref, ln_b_ref, ro_w_ref, ro_b_ref,
                out_ref, memout_ref, mem_all_ref,
                kern_buf, proj_buf, sem_k, sem_p):
    l = pl.program_id(1)
    i = pl.program_id(2)
    mem_ref = mem_all_ref.at[i]
    bb = _BB
    slot = jax.lax.rem(l, 2)

    @pl.when(i == 0)
    def _weights():
        @pl.when(l == 0)
        def _():
            pltpu.make_async_copy(kern_w_ref.at[0], kern_buf.at[0],
                                  sem_k.at[0]).start()
            pltpu.make_async_copy(proj_w_ref.at[0], proj_buf.at[0],
                                  sem_p.at[0]).start()
        pltpu.make_async_copy(kern_w_ref.at[l], kern_buf.at[slot],
                              sem_k.at[slot]).wait()
        pltpu.make_async_copy(proj_w_ref.at[l], proj_buf.at[slot],
                              sem_p.at[slot]).wait()

        @pl.when(l < _DEPTH - 1)
        def _():
            nslot = 1 - slot
            pltpu.make_async_copy(kern_w_ref.at[l + 1], kern_buf.at[nslot],
                                  sem_k.at[nslot]).start()
            pltpu.make_async_copy(proj_w_ref.at[l + 1], proj_buf.at[nslot],
                                  sem_p.at[nslot]).start()

    @pl.when(l == 0)
    def _init():
        comb = inputs_ref[...] + state_ref[...]
        c4 = comb.reshape(bb, _NUMHEADS, _MEMSIZE, _HEADSIZE)
        mem_ref[...] = jnp.concatenate(
            [c4[:, h] for h in range(_NUMHEADS)], axis=-1)

    @pl.when(l == _DEPTH // 2)
    def _mid():
        for p in range(2):
            h4 = mem_ref[:, p * 16:(p + 1) * 16, :].reshape(
                bb, 16, _NUMHEADS, _HEADSIZE)
            mem_ref[:, p * 16:(p + 1) * 16, :] = jnp.concatenate(
                [h4[:, h] for h in range(_NUMHEADS)], axis=-1)

    mem = mem_ref[...]
    mem2d = mem.reshape(bb * _MEMSIZE, _UNITS)
    mean = jnp.mean(mem, axis=1)

    rw = rw_w_ref[0]
    logits2 = jnp.dot(mem2d, rw, preferred_element_type=jnp.float32)
    logits2 = logits2.reshape(bb, _MEMSIZE, 2 * _NUMHEADS)
    corr2 = jnp.dot(mean, rw, preferred_element_type=jnp.float32)

    rlog = logits2[..., :_NUMHEADS] \
        + (corr2[:, :_NUMHEADS] + read_b_ref[0])[:, None, :]
    w = _expand_heads(_softmax_m(rlog), eye_ref)
    att = jnp.sum(w * mem, axis=1)

    v = jnp.maximum(
        jnp.dot(att, kern_buf[slot], preferred_element_type=jnp.float32)
        + kern_b_ref[0], 0.0)
    v = jnp.dot(v, proj_buf[slot], preferred_element_type=jnp.float32) \
        + proj_b_ref[0]
    mu = jnp.mean(v, axis=-1, keepdims=True)
    var = jnp.mean(jnp.square(v - mu), axis=-1, keepdims=True)
    v = (v - mu) * jax.lax.rsqrt(var + _EPS) * ln_g_ref[...] + ln_b_ref[...]

    vcorr = jnp.dot(v, rw[:, _NUMHEADS:],
                    preferred_element_type=jnp.float32)
    wlog = logits2[..., _NUMHEADS:] \
        + (corr2[:, _NUMHEADS:] + vcorr + write_b_ref[0])[:, None, :]
    ww = _expand_heads(_softmax_m(wlog), eye_ref)
    newmem = (1.0 - ww) * mem + ww * v[:, None, :]
    mem_ref[...] = newmem

    @pl.when(l == _DEPTH - 1)
    def _final():
        nm2d = newmem.reshape(bb * _MEMSIZE, _UNITS)
        nmean = jnp.mean(newmem, axis=1)
        logits = jnp.dot(nm2d, ro_w_ref[...],
                         preferred_element_type=jnp.float32)
        corr = jnp.dot(nmean, ro_w_ref[...],
                       preferred_element_type=jnp.float32) + ro_b_ref[...]
        logits = logits.reshape(bb, _MEMSIZE, _NUMHEADS) + corr[:, None, :]
        w = _expand_heads(_softmax_m(logits), eye_ref)
        out_ref[...] = jnp.sum(w * newmem, axis=1)
        for s in range(_MEMSIZE):
            memout_ref[:, s, :] = newmem[:, (s % 2) * 16 + s // 2, :]


def kernel(inputs, state, read_W, read_b, write_W, write_b, kern_W, kern_b,
           proj_W, proj_b, ln_gamma, ln_beta, readout_W, readout_b):
    B = inputs.shape[0]

    rw_W = jnp.concatenate([read_W, write_W], axis=-1)
    read_b3 = read_b.reshape(_DEPTH, 1, _NUMHEADS)
    write_b3 = write_b.reshape(_DEPTH, 1, _NUMHEADS)
    kern_b3 = kern_b.reshape(_DEPTH, 1, _UNITS)
    proj_b3 = proj_b.reshape(_DEPTH, 1, _UNITS)
    ln_g2 = ln_gamma.reshape(1, _UNITS)
    ln_b2 = ln_beta.reshape(1, _UNITS)
    ro_b2 = readout_b.reshape(1, _NUMHEADS)
    eye = jnp.equal(
        jax.lax.broadcasted_iota(jnp.int32, (_NUMHEADS, _UNITS), 1)
        // _HEADSIZE,
        jax.lax.broadcasted_iota(jnp.int32, (_NUMHEADS, _UNITS), 0)
    ).astype(jnp.float32)

    bin_ = lambda o, l, i: (o * _NI + jnp.where(l == 0, i, _NI - 1), 0)
    bout2 = lambda o, l, i: (o * _NI + jnp.where(l == _DEPTH - 1, i, 0), 0)
    bout3 = lambda o, l, i: (o * _NI + jnp.where(l == _DEPTH - 1, i, 0), 0, 0)
    li = lambda o, l, i: (l, 0, 0)
    fixed2 = lambda o, l, i: (0, 0)

    out, memout = pl.pallas_call(
        _dnc_kernel,
        grid=(_NO, _DEPTH, _NI),
        in_specs=[
            pl.BlockSpec((_BB, _MEMSIZE * _UNITS), bin_),
            pl.BlockSpec((_BB, _MEMSIZE * _UNITS), bin_),
            pl.BlockSpec((_NUMHEADS, _UNITS), fixed2),
            pl.BlockSpec((1, _UNITS, 2 * _NUMHEADS), li),
            pl.BlockSpec((1, 1, _NUMHEADS), li),
            pl.BlockSpec((1, 1, _NUMHEADS), li),
            pl.BlockSpec(memory_space=pl.ANY),
            pl.BlockSpec((1, 1, _UNITS), li),
            pl.BlockSpec(memory_space=pl.ANY),
            pl.BlockSpec((1, 1, _UNITS), li),
            pl.BlockSpec((1, _UNITS), fixed2),
            pl.BlockSpec((1, _UNITS), fixed2),
            pl.BlockSpec((_UNITS, _NUMHEADS), fixed2),
            pl.BlockSpec((1, _NUMHEADS), fixed2),
        ],
        out_specs=[
            pl.BlockSpec((_BB, _UNITS), bout2),
            pl.BlockSpec((_BB, _MEMSIZE, _UNITS), bout3),
        ],
        out_shape=[
            jax.ShapeDtypeStruct((B, _UNITS), jnp.float32),
            jax.ShapeDtypeStruct((B, _MEMSIZE, _UNITS), jnp.float32),
        ],
        scratch_shapes=[
            pltpu.VMEM((_NI, _BB, _MEMSIZE, _UNITS), jnp.float32),
            pltpu.VMEM((2, _UNITS, _UNITS), jnp.float32),
            pltpu.VMEM((2, _UNITS, _UNITS), jnp.float32),
            pltpu.SemaphoreType.DMA((2,)),
            pltpu.SemaphoreType.DMA((2,)),
        ],
        compiler_params=pltpu.CompilerParams(
            dimension_semantics=("parallel", "arbitrary", "arbitrary"),
            vmem_limit_bytes=56 * 1024 * 1024,
        ),
        name="dnccell",
    )(inputs, state, eye, rw_W, read_b3, write_b3, kern_W, kern_b3, proj_W,
      proj_b3, ln_g2, ln_b2, readout_W, ro_b2)
    return out, memout.reshape(B, _MEMSIZE * _UNITS)


# speedup vs baseline: 1.4746x; 1.1135x over previous
import jax
import jax.numpy as jnp
from jax.experimental import pallas as pl
from jax.experimental.pallas import tpu as pltpu

_UNITS = 1024
_MEMSIZE = 32
_NUMHEADS = 16
_HEADSIZE = _UNITS // _NUMHEADS
_DEPTH = 8
_EPS = 1e-3
_BB = 16
_NI = 8
_NO = 4


def _softmax_m(logits):
    mx = jnp.max(logits, axis=1, keepdims=True)
    e = jnp.exp(logits - mx)
    return e / jnp.sum(e, axis=1, keepdims=True)


def _expand_heads(w, eye_ref):
    b, m, h = w.shape
    we = jnp.dot(w.reshape(b * m, h), eye_ref[...],
                 preferred_element_type=jnp.float32)
    return we.reshape(b, m, _UNITS)


def _dnc_kernel(inputs_ref, state_ref, eye_ref, rw_ref, ww_ref, wwp_ref,
                read_b_ref, write_bp_ref, kern_w_ref, kern_bp_ref,
                proj_w_ref, proj_bp_ref, ln_g_ref, ln_b_ref, ro_w_ref,
                ro_b_ref, out_ref, memout_ref, mem_all_ref, mean_all_ref,
                att_all_ref, v_all_ref, wlog_all_ref, wc_all_ref,
                kern_buf, proj_buf, sem_k, sem_p):
    t = pl.program_id(1)
    i = pl.program_id(2)
    mem_ref = mem_all_ref.at[i]
    bb = _BB
    pslot = jax.lax.rem(t + 1, 2)

    @pl.when(i == 0)
    def _weights():
        @pl.when(t < _DEPTH)
        def _():
            tslot = jax.lax.rem(t, 2)
            pltpu.make_async_copy(kern_w_ref.at[t], kern_buf.at[tslot],
                                  sem_k.at[tslot]).start()
            pltpu.make_async_copy(proj_w_ref.at[t], proj_buf.at[tslot],
                                  sem_p.at[tslot]).start()

        @pl.when(t >= 1)
        def _():
            tm1 = t - 1
            pltpu.make_async_copy(kern_w_ref.at[tm1], kern_buf.at[pslot],
                                  sem_k.at[pslot]).wait()
            pltpu.make_async_copy(proj_w_ref.at[tm1], proj_buf.at[pslot],
                                  sem_p.at[pslot]).wait()
            att2d = att_all_ref[...].reshape(_NI * _BB, _UNITS)
            vb = jnp.maximum(
                jnp.dot(att2d, kern_buf[pslot],
                        preferred_element_type=jnp.float32)
                + kern_bp_ref[0], 0.0)
            vb = jnp.dot(vb, proj_buf[pslot],
                         preferred_element_type=jnp.float32) + proj_bp_ref[0]
            mu = jnp.mean(vb, axis=-1, keepdims=True)
            var = jnp.mean(jnp.square(vb - mu), axis=-1, keepdims=True)
            vb = (vb - mu) * jax.lax.rsqrt(var + _EPS) * ln_g_ref[...] \
                + ln_b_ref[...]
            v_all_ref[...] = vb.reshape(_NI, _BB, _UNITS)
            mean2d = mean_all_ref[...].reshape(_NI * _BB, _UNITS)
            wc = jnp.dot(mean2d + vb, wwp_ref[0],
                         preferred_element_type=jnp.float32) \
                + write_bp_ref[0]
            wc_all_ref[...] = wc.reshape(_NI, _BB, _NUMHEADS)

    def _apply_update():
        v = v_all_ref[i]
        wlog = wlog_all_ref[i] + wc_all_ref[i][:, None, :]
        ww = _expand_heads(_softmax_m(wlog), eye_ref)
        mem = mem_ref[...]
        return (1.0 - ww) * mem + ww * v[:, None, :]

    @pl.when(t == 0)
    def _t0():
        comb = inputs_ref[...] + state_ref[...]
        c4 = comb.reshape(bb, _NUMHEADS, _MEMSIZE, _HEADSIZE)
        newmem = jnp.concatenate(
            [c4[:, h] for h in range(_NUMHEADS)], axis=-1)
        mem_ref[...] = newmem

    @pl.when((t >= 1) & (t != _DEPTH // 2))
    def _tupd():
        mem_ref[...] = _apply_update()

    @pl.when(t == _DEPTH // 2)
    def _tshuf():
        newmem = _apply_update()
        halves = []
        for p in range(2):
            h4 = newmem[:, p * 16:(p + 1) * 16, :].reshape(
                bb, 16, _NUMHEADS, _HEADSIZE)
            halves.append(jnp.concatenate(
                [h4[:, h] for h in range(_NUMHEADS)], axis=-1))
        mem_ref[...] = jnp.concatenate([halves[0], halves[1]], axis=1)

    @pl.when(t < _DEPTH)
    def _read_phase():
        mem = mem_ref[...]
        mem2d = mem.reshape(bb * _MEMSIZE, _UNITS)
        mean = jnp.mean(mem, axis=1)
        mean_all_ref[i] = mean
        rlog = jnp.dot(mem2d, rw_ref[0],
                       preferred_element_type=jnp.float32)
        rcorr = jnp.dot(mean, rw_ref[0],
                        preferred_element_type=jnp.float32) + read_b_ref[0]
        rlog = rlog.reshape(bb, _MEMSIZE, _NUMHEADS) + rcorr[:, None, :]
        w = _expand_heads(_softmax_m(rlog), eye_ref)
        att_all_ref[i] = jnp.sum(w * mem, axis=1)
        wlog_all_ref[i] = jnp.dot(
            mem2d, ww_ref[0],
            preferred_element_type=jnp.float32).reshape(
                bb, _MEMSIZE, _NUMHEADS)

    @pl.when(t == _DEPTH)
    def _final():
        newmem = mem_ref[...]
        nm2d = newmem.reshape(bb * _MEMSIZE, _UNITS)
        nmean = jnp.mean(newmem, axis=1)
        logits = jnp.dot(nm2d, ro_w_ref[...],
                         preferred_element_type=jnp.float32)
        corr = jnp.dot(nmean, ro_w_ref[...],
                       preferred_element_type=jnp.float32) + ro_b_ref[...]
        logits = logits.reshape(bb, _MEMSIZE, _NUMHEADS) + corr[:, None, :]
        w = _expand_heads(_softmax_m(logits), eye_ref)
        out_ref[...] = jnp.sum(w * newmem, axis=1)
        for s in range(_MEMSIZE):
            memout_ref[:, s, :] = newmem[:, (s % 2) * 16 + s // 2, :]


def kernel(inputs, state, read_W, read_b, write_W, write_b, kern_W, kern_b,
           proj_W, proj_b, ln_gamma, ln_beta, readout_W, readout_b):
    B = inputs.shape[0]

    read_b3 = read_b.reshape(_DEPTH, 1, _NUMHEADS)
    write_b3 = write_b.reshape(_DEPTH, 1, _NUMHEADS)
    kern_b3 = kern_b.reshape(_DEPTH, 1, _UNITS)
    proj_b3 = proj_b.reshape(_DEPTH, 1, _UNITS)
    ln_g2 = ln_gamma.reshape(1, _UNITS)
    ln_b2 = ln_beta.reshape(1, _UNITS)
    ro_b2 = readout_b.reshape(1, _NUMHEADS)
    eye = jnp.equal(
        jax.lax.broadcasted_iota(jnp.int32, (_NUMHEADS, _UNITS), 1)
        // _HEADSIZE,
        jax.lax.broadcasted_iota(jnp.int32, (_NUMHEADS, _UNITS), 0)
    ).astype(jnp.float32)

    bin_ = lambda o, t, i: (o * _NI + jnp.where(t == 0, i, _NI - 1), 0)
    bout2 = lambda o, t, i: (o * _NI + jnp.where(t == _DEPTH, i, 0), 0)
    bout3 = lambda o, t, i: (o * _NI + jnp.where(t == _DEPTH, i, 0), 0, 0)
    lcur = lambda o, t, i: (jnp.minimum(t, _DEPTH - 1), 0, 0)
    lprev = lambda o, t, i: (jnp.maximum(t - 1, 0), 0, 0)
    fixed2 = lambda o, t, i: (0, 0)

    out, memout = pl.pallas_call(
        _dnc_kernel,
        grid=(_NO, _DEPTH + 1, _NI),
        in_specs=[
            pl.BlockSpec((_BB, _MEMSIZE * _UNITS), bin_),
            pl.BlockSpec((_BB, _MEMSIZE * _UNITS), bin_),
            pl.BlockSpec((_NUMHEADS, _UNITS), fixed2),
            pl.BlockSpec((1, _UNITS, _NUMHEADS), lcur),
            pl.BlockSpec((1, _UNITS, _NUMHEADS), lcur),
            pl.BlockSpec((1, _UNITS, _NUMHEADS), lprev),
            pl.BlockSpec((1, 1, _NUMHEADS), lcur),
            pl.BlockSpec((1, 1, _NUMHEADS), lprev),
            pl.BlockSpec(memory_space=pl.ANY),
            pl.BlockSpec((1, 1, _UNITS), lprev),
            pl.BlockSpec(memory_space=pl.ANY),
            pl.BlockSpec((1, 1, _UNITS), lprev),
            pl.BlockSpec((1, _UNITS), fixed2),
            pl.BlockSpec((1, _UNITS), fixed2),
            pl.BlockSpec((_UNITS, _NUMHEADS), fixed2),
            pl.BlockSpec((1, _NUMHEADS), fixed2),
        ],
        out_specs=[
            pl.BlockSpec((_BB, _UNITS), bout2),
            pl.BlockSpec((_BB, _MEMSIZE, _UNITS), bout3),
        ],
        out_shape=[
            jax.ShapeDtypeStruct((B, _UNITS), jnp.float32),
            jax.ShapeDtypeStruct((B, _MEMSIZE, _UNITS), jnp.float32),
        ],
        scratch_shapes=[
            pltpu.VMEM((_NI, _BB, _MEMSIZE, _UNITS), jnp.float32),
            pltpu.VMEM((_NI, _BB, _UNITS), jnp.float32),
            pltpu.VMEM((_NI, _BB, _UNITS), jnp.float32),
            pltpu.VMEM((_NI, _BB, _UNITS), jnp.float32),
            pltpu.VMEM((_NI, _BB, _MEMSIZE, _NUMHEADS), jnp.float32),
            pltpu.VMEM((_NI, _BB, _NUMHEADS), jnp.float32),
            pltpu.VMEM((2, _UNITS, _UNITS), jnp.float32),
            pltpu.VMEM((2, _UNITS, _UNITS), jnp.float32),
            pltpu.SemaphoreType.DMA((2,)),
            pltpu.SemaphoreType.DMA((2,)),
        ],
        compiler_params=pltpu.CompilerParams(
            dimension_semantics=("parallel", "arbitrary", "arbitrary"),
            vmem_limit_bytes=61440 * 1024,
        ),
        name="dnccell",
    )(inputs, state, eye, read_W, write_W, write_W, read_b3, write_b3,
      kern_W, kern_b3, proj_W, proj_b3, ln_g2, ln_b2, readout_W, ro_b2)
    return out, memout.reshape(B, _MEMSIZE * _UNITS)


# speedup vs baseline: 1.6781x; 1.1380x over previous
import jax
import jax.numpy as jnp
from jax.experimental import pallas as pl
from jax.experimental.pallas import tpu as pltpu

_UNITS = 1024
_MEMSIZE = 32
_NUMHEADS = 16
_HEADSIZE = _UNITS // _NUMHEADS
_DEPTH = 8
_EPS = 1e-3
_BB = 16
_NI = 8
_NO = 4


def _softmax_m(logits):
    mx = jnp.max(logits, axis=1, keepdims=True)
    e = jnp.exp(logits - mx)
    return e / jnp.sum(e, axis=1, keepdims=True)


def _expand_heads(w, eye_ref):
    b, m, h = w.shape
    we = jnp.dot(w.reshape(b * m, h), eye_ref[...],
                 preferred_element_type=jnp.float32)
    return we.reshape(b, m, _UNITS)


def _dnc_kernel(inputs_ref, state_ref, eye_ref, rw_ref, ww_ref, wwp_ref,
                read_b_ref, write_bp_ref, kern_w_ref, kern_bp_ref,
                proj_w_ref, proj_bp_ref, ln_g_ref, ln_b_ref, ro_w_ref,
                ro_b_ref, out_ref, memout_ref, mem_all_ref, mean_all_ref,
                att_all_ref, v_all_ref, wc_all_ref,
                kern_buf, proj_buf, sem_k, sem_p):
    t = pl.program_id(1)
    i = pl.program_id(2)
    mem_ref = mem_all_ref.at[i]
    bb = _BB
    pslot = jax.lax.rem(t + 1, 2)

    @pl.when(i == 0)
    def _weights():
        @pl.when(t < _DEPTH)
        def _():
            tslot = jax.lax.rem(t, 2)
            pltpu.make_async_copy(kern_w_ref.at[t], kern_buf.at[tslot],
                                  sem_k.at[tslot]).start()
            pltpu.make_async_copy(proj_w_ref.at[t], proj_buf.at[tslot],
                                  sem_p.at[tslot]).start()

        @pl.when(t >= 1)
        def _():
            tm1 = t - 1
            pltpu.make_async_copy(kern_w_ref.at[tm1], kern_buf.at[pslot],
                                  sem_k.at[pslot]).wait()
            pltpu.make_async_copy(proj_w_ref.at[tm1], proj_buf.at[pslot],
                                  sem_p.at[pslot]).wait()
            att2d = att_all_ref[...].reshape(_NI * _BB, _UNITS)
            vb = jnp.maximum(
                jnp.dot(att2d, kern_buf[pslot],
                        preferred_element_type=jnp.float32)
                + kern_bp_ref[0], 0.0)
            vb = jnp.dot(vb, proj_buf[pslot],
                         preferred_element_type=jnp.float32) + proj_bp_ref[0]
            mu = jnp.mean(vb, axis=-1, keepdims=True)
            var = jnp.mean(jnp.square(vb - mu), axis=-1, keepdims=True)
            vb = (vb - mu) * jax.lax.rsqrt(var + _EPS) * ln_g_ref[...] \
                + ln_b_ref[...]
            v_all_ref[...] = vb.reshape(_NI, _BB, _UNITS)
            mean2d = mean_all_ref[...].reshape(_NI * _BB, _UNITS)
            wc = jnp.dot(mean2d + vb, wwp_ref[0],
                         preferred_element_type=jnp.float32) \
                + write_bp_ref[0]
            wc_all_ref[...] = wc.reshape(_NI, _BB, _NUMHEADS)

    def _apply_update():
        v = v_all_ref[i]
        mem = mem_ref[...]
        wlogp = jnp.dot(mem.reshape(bb * _MEMSIZE, _UNITS), wwp_ref[0],
                        preferred_element_type=jnp.float32)
        wlog = wlogp.reshape(bb, _MEMSIZE, _NUMHEADS) \
            + wc_all_ref[i][:, None, :]
        ww = _expand_heads(_softmax_m(wlog), eye_ref)
        return (1.0 - ww) * mem + ww * v[:, None, :]

    @pl.when(t == 0)
    def _t0():
        comb = inputs_ref[...] + state_ref[...]
        c4 = comb.reshape(bb, _NUMHEADS, _MEMSIZE, _HEADSIZE)
        newmem = jnp.concatenate(
            [c4[:, h] for h in range(_NUMHEADS)], axis=-1)
        mem_ref[...] = newmem

    @pl.when((t >= 1) & (t != _DEPTH // 2))
    def _tupd():
        mem_ref[...] = _apply_update()

    @pl.when(t == _DEPTH // 2)
    def _tshuf():
        newmem = _apply_update()
        halves = []
        for p in range(2):
            h4 = newmem[:, p * 16:(p + 1) * 16, :].reshape(
                bb, 16, _NUMHEADS, _HEADSIZE)
            halves.append(jnp.concatenate(
                [h4[:, h] for h in range(_NUMHEADS)], axis=-1))
        mem_ref[...] = jnp.concatenate([halves[0], halves[1]], axis=1)

    @pl.when(t < _DEPTH)
    def _read_phase():
        mem = mem_ref[...]
        mem2d = mem.reshape(bb * _MEMSIZE, _UNITS)
        mean = jnp.mean(mem, axis=1)
        mean_all_ref[i] = mean
        rlog = jnp.dot(mem2d, rw_ref[0],
                       preferred_element_type=jnp.float32)
        rcorr = jnp.dot(mean, rw_ref[0],
                        preferred_element_type=jnp.float32) + read_b_ref[0]
        rlog = rlog.reshape(bb, _MEMSIZE, _NUMHEADS) + rcorr[:, None, :]
        w = _expand_heads(_softmax_m(rlog), eye_ref)
        att_all_ref[i] = jnp.sum(w * mem, axis=1)

    @pl.when(t == _DEPTH)
    def _final():
        newmem = mem_ref[...]
        nm2d = newmem.reshape(bb * _MEMSIZE, _UNITS)
        nmean = jnp.mean(newmem, axis=1)
        logits = jnp.dot(nm2d, ro_w_ref[...],
                         preferred_element_type=jnp.float32)
        corr = jnp.dot(nmean, ro_w_ref[...],
                       preferred_element_type=jnp.float32) + ro_b_ref[...]
        logits = logits.reshape(bb, _MEMSIZE, _NUMHEADS) + corr[:, None, :]
        w = _expand_heads(_softmax_m(logits), eye_ref)
        out_ref[...] = jnp.sum(w * newmem, axis=1)
        for s in range(_MEMSIZE):
            memout_ref[:, s, :] = newmem[:, (s % 2) * 16 + s // 2, :]


def kernel(inputs, state, read_W, read_b, write_W, write_b, kern_W, kern_b,
           proj_W, proj_b, ln_gamma, ln_beta, readout_W, readout_b):
    B = inputs.shape[0]

    read_b3 = read_b.reshape(_DEPTH, 1, _NUMHEADS)
    write_b3 = write_b.reshape(_DEPTH, 1, _NUMHEADS)
    kern_b3 = kern_b.reshape(_DEPTH, 1, _UNITS)
    proj_b3 = proj_b.reshape(_DEPTH, 1, _UNITS)
    ln_g2 = ln_gamma.reshape(1, _UNITS)
    ln_b2 = ln_beta.reshape(1, _UNITS)
    ro_b2 = readout_b.reshape(1, _NUMHEADS)
    eye = jnp.equal(
        jax.lax.broadcasted_iota(jnp.int32, (_NUMHEADS, _UNITS), 1)
        // _HEADSIZE,
        jax.lax.broadcasted_iota(jnp.int32, (_NUMHEADS, _UNITS), 0)
    ).astype(jnp.float32)

    bin_ = lambda o, t, i: (o * _NI + jnp.where(t == 0, i, _NI - 1), 0)
    bout2 = lambda o, t, i: (o * _NI + jnp.where(t == _DEPTH, i, 0), 0)
    bout3 = lambda o, t, i: (o * _NI + jnp.where(t == _DEPTH, i, 0), 0, 0)
    lcur = lambda o, t, i: (jnp.minimum(t, _DEPTH - 1), 0, 0)
    lprev = lambda o, t, i: (jnp.maximum(t - 1, 0), 0, 0)
    fixed2 = lambda o, t, i: (0, 0)

    out, memout = pl.pallas_call(
        _dnc_kernel,
        grid=(_NO, _DEPTH + 1, _NI),
        in_specs=[
            pl.BlockSpec((_BB, _MEMSIZE * _UNITS), bin_),
            pl.BlockSpec((_BB, _MEMSIZE * _UNITS), bin_),
            pl.BlockSpec((_NUMHEADS, _UNITS), fixed2),
            pl.BlockSpec((1, _UNITS, _NUMHEADS), lcur),
            pl.BlockSpec((1, _UNITS, _NUMHEADS), lcur),
            pl.BlockSpec((1, _UNITS, _NUMHEADS), lprev),
            pl.BlockSpec((1, 1, _NUMHEADS), lcur),
            pl.BlockSpec((1, 1, _NUMHEADS), lprev),
            pl.BlockSpec(memory_space=pl.ANY),
            pl.BlockSpec((1, 1, _UNITS), lprev),
            pl.BlockSpec(memory_space=pl.ANY),
            pl.BlockSpec((1, 1, _UNITS), lprev),
            pl.BlockSpec((1, _UNITS), fixed2),
            pl.BlockSpec((1, _UNITS), fixed2),
            pl.BlockSpec((_UNITS, _NUMHEADS), fixed2),
            pl.BlockSpec((1, _NUMHEADS), fixed2),
        ],
        out_specs=[
            pl.BlockSpec((_BB, _UNITS), bout2),
            pl.BlockSpec((_BB, _MEMSIZE, _UNITS), bout3),
        ],
        out_shape=[
            jax.ShapeDtypeStruct((B, _UNITS), jnp.float32),
            jax.ShapeDtypeStruct((B, _MEMSIZE, _UNITS), jnp.float32),
        ],
        scratch_shapes=[
            pltpu.VMEM((_NI, _BB, _MEMSIZE, _UNITS), jnp.float32),
            pltpu.VMEM((_NI, _BB, _UNITS), jnp.float32),
            pltpu.VMEM((_NI, _BB, _UNITS), jnp.float32),
            pltpu.VMEM((_NI, _BB, _UNITS), jnp.float32),
            pltpu.VMEM((_NI, _BB, _NUMHEADS), jnp.float32),
            pltpu.VMEM((2, _UNITS, _UNITS), jnp.float32),
            pltpu.VMEM((2, _UNITS, _UNITS), jnp.float32),
            pltpu.SemaphoreType.DMA((2,)),
            pltpu.SemaphoreType.DMA((2,)),
        ],
        compiler_params=pltpu.CompilerParams(
            dimension_semantics=("parallel", "arbitrary", "arbitrary"),
            vmem_limit_bytes=61440 * 1024,
        ),
        name="dnccell",
    )(inputs, state, eye, read_W, write_W, write_W, read_b3, write_b3,
      kern_W, kern_b3, proj_W, proj_b3, ln_g2, ln_b2, readout_W, ro_b2)
    return out, memout.reshape(B, _MEMSIZE * _UNITS)
